# baseline (speedup 1.0000x reference)
import jax
import jax.numpy as jnp
from jax import lax
from jax.experimental import pallas as pl
from jax.experimental.pallas import tpu as pltpu
from jax.experimental.pallas import tpu_sc as plsc

B, T = 2, 2048
D, H, KH = 768, 12, 6
DH = 64
E, TOPK, F = 8, 2, 512
NEG = -1e30
TB = 512
NTB = (B * T) // TB
CH = 512
BE = 512
NBLK = (B * T * TOPK) // BE + E
NROWS = NBLK * BE
NW = 32
BPW = (B * T) // NW

_f32 = jnp.float32
_bf = jnp.bfloat16


def _ln_body(x, g, b):
    mu = jnp.mean(x, axis=-1, keepdims=True)
    xc = x - mu
    var = jnp.mean(xc * xc, axis=-1, keepdims=True)
    return xc * lax.rsqrt(var + 1e-5) * g + b


def _rot(t):
    half = t.shape[-1] // 2
    return jnp.concatenate([-t[:, half:], t[:, :half]], axis=-1)


def _bp_body(x_ref, g1_ref, b1_ref, wrpt_ref, wr_ref,
             wq_ref, wk_ref, wv_ref, wo_ref, cos_ref, sin_ref,
             x1_ref, hbf_ref, ti_ref, tw_ref):
    kk = pl.program_id(1)
    cos = cos_ref[...]
    sin = sin_ref[...]

    @pl.when(kk == 0)
    def _():
        pooled = jnp.zeros((1, DH), _f32)
        for c in range(T // CH):
            sl = slice(c * CH, (c + 1) * CH)
            hc = _ln_body(x_ref[0, sl, :], g1_ref[0], b1_ref[0])
            hbf_ref[sl, :] = hc
            xr = jnp.dot(hc, wrpt_ref[...], preferred_element_type=_f32)
            xr = xr * cos[sl] + _rot(xr) * sin[sl]
            pooled = pooled + jnp.sum(xr, axis=0, keepdims=True)
        logits = jnp.dot(pooled * (1.0 / T), wr_ref[...],
                         preferred_element_type=_f32)
        it = lax.broadcasted_iota(jnp.int32, (1, H), 1)
        l = logits
        tvs, tis = [], []
        for _ in range(KH):
            m = jnp.max(l)
            i = jnp.min(jnp.where(l == m, it, H))
            tvs.append(m)
            tis.append(i)
            l = jnp.where(it == i, NEG, l)
        exps = [jnp.exp(v - tvs[0]) for v in tvs]
        denom = exps[0]
        for e_ in exps[1:]:
            denom = denom + e_
        for j in range(KH):
            ti_ref[j] = tis[j]
            tw_ref[j] = exps[j] / denom
        x1_ref[0] = x_ref[0]

    idx = ti_ref[kk]
    wgt = tw_ref[kk]
    hbf = hbf_ref[...]
    scale = DH ** -0.5
    q = jnp.dot(hbf, wq_ref[idx], preferred_element_type=_f32)
    k = jnp.dot(hbf, wk_ref[idx], preferred_element_type=_f32)
    v = jnp.dot(hbf, wv_ref[idx], preferred_element_type=_f32)
    q = q * cos + _rot(q) * sin
    k = k * cos + _rot(k) * sin
    wo = wo_ref[idx]
    vaug = jnp.concatenate([v, jnp.ones((T, DH), _f32)], axis=1)
    tri = (lax.broadcasted_iota(jnp.int32, (CH, CH), 1)
           > lax.broadcasted_iota(jnp.int32, (CH, CH), 0))
    for i in range(T // CH):
        kl = (i + 1) * CH
        qc = q[i * CH:kl] * scale
        s = lax.dot_general(qc, k[:kl], (((1,), (1,)), ((), ())),
                            preferred_element_type=_f32)
        pd = jnp.where(tri, 0.0, jnp.exp(s[:, i * CH:kl]))
        if i == 0:
            p = pd
        else:
            p = jnp.concatenate([jnp.exp(s[:, :i * CH]), pd], axis=1)
        ctxa = jnp.dot(p, vaug[:kl], preferred_element_type=_f32)
        ctx = ctxa[:, :DH] / ctxa[:, DH:DH + 1]
        oph = jnp.dot(ctx, wo, preferred_element_type=_f32)
        x1_ref[0, i * CH:kl, :] += oph * wgt


def _c0_body(x1_ref, g_ref, b_ref, wrt_ref, h2_ref, fw_ref, s1_ref, s2_ref):
    h2 = _ln_body(x1_ref[...], g_ref[0], b_ref[0])
    h2_ref[...] = h2
    rl = jnp.dot(h2, wrt_ref[...], preferred_element_type=_f32)
    it = lax.broadcasted_iota(jnp.int32, (TB, E), 1)
    m1 = jnp.max(rl, axis=1, keepdims=True)
    i1 = jnp.min(jnp.where(rl == m1, it, E), axis=1, keepdims=True)
    rl2 = jnp.where(it == i1, NEG, rl)
    m2 = jnp.max(rl2, axis=1, keepdims=True)
    i2 = jnp.min(jnp.where(rl2 == m2, it, E), axis=1, keepdims=True)
    w1 = 1.0 / (1.0 + jnp.exp(m2 - m1))
    w2 = 1.0 - w1
    sel1 = jnp.where(it == i1, 1.0, 0.0)
    sel2 = jnp.where(it == i2, 1.0, 0.0)
    s1_ref[...] = sel1
    s2_ref[...] = sel2
    fw_ref[...] = sel1 * w1 + sel2 * w2


def _d0_body(s1_ref, s2_ref, s12_ref, bexp_ref):
    sel1 = s1_ref[...]
    sel2 = s2_ref[...]
    oh = sel1 + sel2
    counts = jnp.sum(oh, axis=0, keepdims=True)
    ci = counts.astype(jnp.int32)
    cp = jnp.bitwise_and(ci + (BE - 1), ~(BE - 1))
    cpf = cp.astype(_f32)
    r8 = lax.broadcasted_iota(jnp.int32, (E, E), 0)
    c8 = lax.broadcasted_iota(jnp.int32, (E, E), 1)
    useg = jnp.where(r8 < c8, 1.0, 0.0)
    seg = jnp.dot(cpf, useg, preferred_element_type=_f32)
    rT = lax.broadcasted_iota(jnp.int32, (TB, TB), 0)
    cT = lax.broadcasted_iota(jnp.int32, (TB, TB), 1)
    tril = jnp.where(rT > cT, 1.0, 0.0)
    base = jnp.zeros((1, E), _f32)
    i2c = lax.broadcasted_iota(jnp.int32, (TB, 2), 1)
    for c in range((B * T) // TB):
        sl = slice(c * TB, (c + 1) * TB)
        ohc = oh[sl]
        pos = jnp.dot(tril, ohc, preferred_element_type=_f32) + base + seg
        base = base + jnp.sum(ohc, axis=0, keepdims=True)
        s1c = jnp.sum(pos * sel1[sl], axis=1, keepdims=True)
        s2c = jnp.sum(pos * sel2[sl], axis=1, keepdims=True)
        s12_ref[sl, :] = jnp.where(i2c == 0, s1c, s2c).astype(jnp.int32)
    segend = seg + cpf
    lane = lax.broadcasted_iota(jnp.int32, (1, 128), 1) * BE
    acc = jnp.zeros((1, 128), jnp.int32)
    for e in range(E):
        acc = acc + jnp.where(lane >= segend[0, e].astype(jnp.int32), 1, 0)
    bexp_ref[...] = jnp.minimum(acc, E - 1)


def _d1_body(h2_hbm, s1_hbm, s2_hbm, grp_hbm, idx_v, rows_v, sem):
    wid = lax.axis_index("s") * 2 + lax.axis_index("c")
    base = wid * BPW
    pltpu.sync_copy(h2_hbm.at[pl.ds(base, BPW)], rows_v)
    pltpu.sync_copy(s1_hbm.at[pl.ds(base, BPW)], idx_v)
    pltpu.async_copy(rows_v, grp_hbm.at[idx_v], sem).wait()
    pltpu.sync_copy(s2_hbm.at[pl.ds(base, BPW)], idx_v)
    pltpu.async_copy(rows_v, grp_hbm.at[idx_v], sem).wait()


def _c1s_body(bexp_ref, grp_ref, w1_ref, w3_ref, w2_ref, eo_ref):
    j = pl.program_id(0)
    e = bexp_ref[j]
    g = grp_ref[...].astype(_bf)
    h1 = jnp.dot(g, w1_ref[e], preferred_element_type=_f32)
    h3 = jnp.dot(g, w3_ref[e], preferred_element_type=_f32)
    he = (h1 * (1.0 / (1.0 + jnp.exp(-h1))) * h3).astype(_bf)
    eo_ref[...] = jnp.dot(he, w2_ref[e], preferred_element_type=_f32)


def _d2_body(eo_hbm, s1_hbm, s2_hbm, eo1_hbm, eo2_hbm, idx_v, rows_v, sem):
    wid = lax.axis_index("s") * 2 + lax.axis_index("c")
    base = wid * BPW
    pltpu.sync_copy(s1_hbm.at[pl.ds(base, BPW)], idx_v)
    pltpu.async_copy(eo_hbm.at[idx_v], rows_v, sem).wait()
    pltpu.sync_copy(rows_v, eo1_hbm.at[pl.ds(base, BPW)])
    pltpu.sync_copy(s2_hbm.at[pl.ds(base, BPW)], idx_v)
    pltpu.async_copy(eo_hbm.at[idx_v], rows_v, sem).wait()
    pltpu.sync_copy(rows_v, eo2_hbm.at[pl.ds(base, BPW)])


def _f_body(x1_ref, e1_ref, e2_ref, fw_ref, s1_ref, s2_ref, out_ref):
    w1 = jnp.sum(fw_ref[...] * s1_ref[...], axis=1, keepdims=True)
    w2 = jnp.sum(fw_ref[...] * s2_ref[...], axis=1, keepdims=True)
    out_ref[...] = x1_ref[...] + w1 * e1_ref[...] + w2 * e2_ref[...]


def kernel(x, causal_mask, attention_mask, positions, ln1_g, ln1_b, ln2_g, ln2_b,
           Wrp, Wr, Wq, Wk, Wv, Wo, Wrouter, W1, W2, W3):
    half = DH // 2
    inv_freq = 1.0 / (10000.0 ** (jnp.arange(half, dtype=_f32) * 2.0 / DH))
    ang = positions.astype(_f32)[:, None] * inv_freq[None, :]
    cos = jnp.concatenate([jnp.cos(ang), jnp.cos(ang)], axis=-1)
    sin = jnp.concatenate([jnp.sin(ang), jnp.sin(ang)], axis=-1)

    g1 = ln1_g.reshape(1, D)
    b1 = ln1_b.reshape(1, D)
    g2 = ln2_g.reshape(1, D)
    b2 = ln2_b.reshape(1, D)

    full2 = lambda b, k: (0, 0)
    full3 = lambda b, k: (0, 0, 0)
    blk = lambda b, k: (b, 0, 0)
    x1 = pl.pallas_call(
        _bp_body,
        grid=(B, KH),
        in_specs=[
            pl.BlockSpec((1, T, D), blk),
            pl.BlockSpec((1, D), full2),
            pl.BlockSpec((1, D), full2),
            pl.BlockSpec((D, DH), full2),
            pl.BlockSpec((DH, H), full2),
            pl.BlockSpec((H, D, DH), full3),
            pl.BlockSpec((H, D, DH), full3),
            pl.BlockSpec((H, D, DH), full3),
            pl.BlockSpec((H, DH, D), full3),
            pl.BlockSpec((T, DH), full2),
            pl.BlockSpec((T, DH), full2),
        ],
        out_specs=pl.BlockSpec((1, T, D), blk),
        out_shape=jax.ShapeDtypeStruct((B, T, D), _f32),
        scratch_shapes=[pltpu.VMEM((T, D), _f32),
                        pltpu.SMEM((KH,), jnp.int32),
                        pltpu.SMEM((KH,), _f32)],
        compiler_params=pltpu.CompilerParams(
            dimension_semantics=("arbitrary", "arbitrary")),
    )(x, g1, b1, Wrp.T, Wr.T, Wq, Wk, Wv, Wo, cos, sin)

    x12 = x1.reshape(B * T, D)
    h2, fw, sel1, sel2 = pl.pallas_call(
        _c0_body,
        grid=(NTB,),
        in_specs=[
            pl.BlockSpec((TB, D), lambda t: (t, 0)),
            pl.BlockSpec((1, D), lambda t: (0, 0)),
            pl.BlockSpec((1, D), lambda t: (0, 0)),
            pl.BlockSpec((D, E), lambda t: (0, 0)),
        ],
        out_specs=[
            pl.BlockSpec((TB, D), lambda t: (t, 0)),
            pl.BlockSpec((TB, E), lambda t: (t, 0)),
            pl.BlockSpec((TB, E), lambda t: (t, 0)),
            pl.BlockSpec((TB, E), lambda t: (t, 0)),
        ],
        out_shape=[
            jax.ShapeDtypeStruct((B * T, D), _f32),
            jax.ShapeDtypeStruct((B * T, E), _f32),
            jax.ShapeDtypeStruct((B * T, E), _f32),
            jax.ShapeDtypeStruct((B * T, E), _f32),
        ],
    )(x12, g2, b2, Wrouter.T)

    s12, bexp = pl.pallas_call(
        _d0_body,
        out_shape=[
            jax.ShapeDtypeStruct((B * T, 2), jnp.int32),
            jax.ShapeDtypeStruct((1, 128), jnp.int32),
        ],
    )(sel1, sel2)
    s1i = s12[:, 0]
    s2i = s12[:, 1]

    mesh = plsc.VectorSubcoreMesh(core_axis_name="c", subcore_axis_name="s")
    grp = pl.kernel(
        _d1_body,
        mesh=mesh,
        out_type=jax.ShapeDtypeStruct((NROWS, D), _f32),
        scratch_types=[
            pltpu.VMEM((BPW,), jnp.int32),
            pltpu.VMEM((BPW, D), _f32),
            pltpu.SemaphoreType.DMA,
        ],
    )(h2, s1i, s2i)

    eog = pl.pallas_call(
        _c1s_body,
        grid_spec=pltpu.PrefetchScalarGridSpec(
            num_scalar_prefetch=1,
            grid=(NBLK,),
            in_specs=[
                pl.BlockSpec((BE, D), lambda j, bexp: (j, 0)),
                pl.BlockSpec((E, D, F), lambda j, bexp: (0, 0, 0)),
                pl.BlockSpec((E, D, F), lambda j, bexp: (0, 0, 0)),
                pl.BlockSpec((E, F, D), lambda j, bexp: (0, 0, 0)),
            ],
            out_specs=pl.BlockSpec((BE, D), lambda j, bexp: (j, 0)),
        ),
        out_shape=jax.ShapeDtypeStruct((NROWS, D), _f32),
        compiler_params=pltpu.CompilerParams(
            dimension_semantics=("arbitrary",)),
    )(bexp.reshape(128), grp, W1.astype(_bf), W3.astype(_bf), W2.astype(_bf))

    eo1, eo2 = pl.kernel(
        _d2_body,
        mesh=mesh,
        out_type=[
            jax.ShapeDtypeStruct((B * T, D), _f32),
            jax.ShapeDtypeStruct((B * T, D), _f32),
        ],
        scratch_types=[
            pltpu.VMEM((BPW,), jnp.int32),
            pltpu.VMEM((BPW, D), _f32),
            pltpu.SemaphoreType.DMA,
        ],
    )(eog, s1i, s2i)

    out = pl.pallas_call(
        _f_body,
        grid=(NTB,),
        in_specs=[
            pl.BlockSpec((TB, D), lambda t: (t, 0)),
            pl.BlockSpec((TB, D), lambda t: (t, 0)),
            pl.BlockSpec((TB, D), lambda t: (t, 0)),
            pl.BlockSpec((TB, E), lambda t: (t, 0)),
            pl.BlockSpec((TB, E), lambda t: (t, 0)),
            pl.BlockSpec((TB, E), lambda t: (t, 0)),
        ],
        out_specs=pl.BlockSpec((TB, D), lambda t: (t, 0)),
        out_shape=jax.ShapeDtypeStruct((B * T, D), _f32),
    )(x12, eo1, eo2, fw, sel1, sel2)

    return out.reshape(B, T, D)
